# initial kernel scaffold (unmeasured)
import jax
import jax.numpy as jnp
from jax import lax
from jax.experimental import pallas as pl
from jax.experimental.pallas import tpu as pltpu

N_DEV = 8
N_ROUNDS = 3
B, SQ, D = 1, 512, 1024
HQ, HKV, DH = 8, 2, 128
GROUP = HQ // HKV
SCALE = 0.08838834764831843


def kernel(x, Wq, Wo, K_ext, V_ext):
    def body(x_ref, wq_ref, wo_ref, k_ref, v_ref, out_ref,
             state_o, state_ml, comm_o, comm_ml,
             send_o, recv_o, send_ml, recv_ml):
        my = lax.axis_index("i")

        barrier = pltpu.get_barrier_semaphore()
        for r in range(N_ROUNDS):
            partner = jnp.bitwise_xor(my, 1 << r)
            pl.semaphore_signal(barrier, inc=1, device_id=(partner,),
                                device_id_type=pl.DeviceIdType.MESH)
        pl.semaphore_wait(barrier, N_ROUNDS)

        q = jnp.dot(x_ref[0], wq_ref[:, :],
                    preferred_element_type=jnp.float32)

        for h in range(HQ):
            qh = q[:, h * DH:(h + 1) * DH]
            kh = k_ref[0, :, h // GROUP, :]
            vh = v_ref[0, :, h // GROUP, :]
            s = lax.dot_general(qh, kh, (((1,), (1,)), ((), ())),
                                preferred_element_type=jnp.float32) * SCALE
            mh = jnp.max(s, axis=1, keepdims=True)
            p = jnp.exp(s - mh)
            lh = jnp.sum(p, axis=1, keepdims=True)
            oh = lax.dot_general(p, vh, (((1,), (0,)), ((), ())),
                                 preferred_element_type=jnp.float32)
            state_o[h, :, :] = oh
            state_ml[0, h, :] = mh[:, 0]
            state_ml[1, h, :] = lh[:, 0]

        for r in range(N_ROUNDS):
            partner = jnp.bitwise_xor(my, 1 << r)
            rdma_o = pltpu.make_async_remote_copy(
                src_ref=state_o, dst_ref=comm_o.at[r],
                send_sem=send_o.at[r], recv_sem=recv_o.at[r],
                device_id=(partner,), device_id_type=pl.DeviceIdType.MESH)
            rdma_ml = pltpu.make_async_remote_copy(
                src_ref=state_ml, dst_ref=comm_ml.at[r],
                send_sem=send_ml.at[r], recv_sem=recv_ml.at[r],
                device_id=(partner,), device_id_type=pl.DeviceIdType.MESH)
            rdma_o.start()
            rdma_ml.start()
            rdma_o.wait()
            rdma_ml.wait()

            m_a = state_ml[0, :, :]
            l_a = state_ml[1, :, :]
            o_a = state_o[:, :, :]
            m_b = comm_ml[r, 0, :, :]
            l_b = comm_ml[r, 1, :, :]
            o_b = comm_o[r, :, :, :]
            m_n = jnp.maximum(m_a, m_b)
            a = jnp.exp(m_a - m_n)
            b = jnp.exp(m_b - m_n)
            state_ml[0, :, :] = m_n
            state_ml[1, :, :] = a * l_a + b * l_b
            state_o[:, :, :] = (a[:, :, None] * o_a + b[:, :, None] * o_b)

        acc = jnp.zeros((SQ, D), jnp.float32)
        for h in range(HQ):
            o_h = state_o[h, :, :] / state_ml[1, h, :][:, None]
            acc = acc + jnp.dot(o_h, wo_ref[h * DH:(h + 1) * DH, :],
                                preferred_element_type=jnp.float32)
        out_ref[0, :, :] = acc

    return pl.pallas_call(
        body,
        out_shape=jax.ShapeDtypeStruct((B, SQ, D), jnp.float32),
        in_specs=[pl.BlockSpec(memory_space=pltpu.VMEM)] * 5,
        out_specs=pl.BlockSpec(memory_space=pltpu.VMEM),
        scratch_shapes=[
            pltpu.VMEM((HQ, SQ, DH), jnp.float32),
            pltpu.VMEM((2, HQ, SQ), jnp.float32),
            pltpu.VMEM((N_ROUNDS, HQ, SQ, DH), jnp.float32),
            pltpu.VMEM((N_ROUNDS, 2, HQ, SQ), jnp.float32),
            pltpu.SemaphoreType.DMA((N_ROUNDS,)),
            pltpu.SemaphoreType.DMA((N_ROUNDS,)),
            pltpu.SemaphoreType.DMA((N_ROUNDS,)),
            pltpu.SemaphoreType.DMA((N_ROUNDS,)),
        ],
        compiler_params=pltpu.CompilerParams(collective_id=0),
    )(x, Wq, Wo, K_ext, V_ext)


# baseline (device time: 120081 ns/iter reference)
import jax
import jax.numpy as jnp
from jax import lax
from jax.experimental import pallas as pl
from jax.experimental.pallas import tpu as pltpu

N_DEV = 8
N_ROUNDS = 3
B, SQ, D = 1, 512, 1024
HQ, HKV, DH = 8, 2, 128
GROUP = HQ // HKV
SCALE = 0.08838834764831843


def kernel(x, Wq, Wo, K_ext, V_ext):
    def body(x_ref, wq_ref, wo_ref, k_ref, v_ref, out_ref,
             state_o, state_ml, comm_o, comm_ml,
             send_o, recv_o, send_ml, recv_ml):
        my = lax.axis_index("i")

        barrier = pltpu.get_barrier_semaphore()
        for r in range(N_ROUNDS):
            partner = jnp.bitwise_xor(my, 1 << r)
            pl.semaphore_signal(barrier, inc=1, device_id=(partner,),
                                device_id_type=pl.DeviceIdType.MESH)
        pl.semaphore_wait(barrier, N_ROUNDS)

        q = jnp.dot(x_ref[0], wq_ref[:, :],
                    preferred_element_type=jnp.float32)

        for h in range(HQ):
            qh = q[:, h * DH:(h + 1) * DH]
            kh = k_ref[0, :, h // GROUP, :]
            vh = v_ref[0, :, h // GROUP, :]
            s = lax.dot_general(qh, kh, (((1,), (1,)), ((), ())),
                                preferred_element_type=jnp.float32) * SCALE
            mh = jnp.max(s, axis=1, keepdims=True)
            p = jnp.exp(s - mh)
            lh = jnp.sum(p, axis=1, keepdims=True)
            oh = lax.dot_general(p, vh, (((1,), (0,)), ((), ())),
                                 preferred_element_type=jnp.float32)
            state_o[h, :, :] = oh
            state_ml[0, h, :] = mh[:, 0]
            state_ml[1, h, :] = lh[:, 0]

        for r in range(N_ROUNDS):
            partner = jnp.bitwise_xor(my, 1 << r)
            rdma_o = pltpu.make_async_remote_copy(
                src_ref=state_o, dst_ref=comm_o.at[r],
                send_sem=send_o.at[r], recv_sem=recv_o.at[r],
                device_id=(partner,), device_id_type=pl.DeviceIdType.MESH)
            rdma_ml = pltpu.make_async_remote_copy(
                src_ref=state_ml, dst_ref=comm_ml.at[r],
                send_sem=send_ml.at[r], recv_sem=recv_ml.at[r],
                device_id=(partner,), device_id_type=pl.DeviceIdType.MESH)
            rdma_o.start()
            rdma_ml.start()
            rdma_o.wait()
            rdma_ml.wait()

            m_a = state_ml[0, :, :]
            l_a = state_ml[1, :, :]
            o_a = state_o[:, :, :]
            m_b = comm_ml[r, 0, :, :]
            l_b = comm_ml[r, 1, :, :]
            o_b = comm_o[r, :, :, :]
            m_n = jnp.maximum(m_a, m_b)
            a = jnp.exp(m_a - m_n)
            b = jnp.exp(m_b - m_n)
            state_ml[0, :, :] = m_n
            state_ml[1, :, :] = a * l_a + b * l_b
            state_o[:, :, :] = (a[:, :, None] * o_a + b[:, :, None] * o_b)

        acc = jnp.zeros((SQ, D), jnp.float32)
        for h in range(HQ):
            o_h = state_o[h, :, :] / state_ml[1, h, :][:, None]
            acc = acc + jnp.dot(o_h, wo_ref[h * DH:(h + 1) * DH, :],
                                preferred_element_type=jnp.float32)
        out_ref[0, :, :] = acc

    return pl.pallas_call(
        body,
        out_shape=jax.ShapeDtypeStruct((B, SQ, D), jnp.float32),
        in_specs=[pl.BlockSpec(memory_space=pltpu.VMEM)] * 5,
        out_specs=pl.BlockSpec(memory_space=pltpu.VMEM),
        scratch_shapes=[
            pltpu.VMEM((HQ, SQ, DH), jnp.float32),
            pltpu.VMEM((2, HQ, SQ), jnp.float32),
            pltpu.VMEM((N_ROUNDS, HQ, SQ, DH), jnp.float32),
            pltpu.VMEM((N_ROUNDS, 2, HQ, SQ), jnp.float32),
            pltpu.SemaphoreType.DMA((N_ROUNDS,)),
            pltpu.SemaphoreType.DMA((N_ROUNDS,)),
            pltpu.SemaphoreType.DMA((N_ROUNDS,)),
            pltpu.SemaphoreType.DMA((N_ROUNDS,)),
        ],
        compiler_params=pltpu.CompilerParams(
            collective_id=0, vmem_limit_bytes=64 * 1024 * 1024),
    )(x, Wq, Wo, K_ext, V_ext)


# device time: 86393 ns/iter; 1.3899x vs baseline; 1.3899x over previous
import jax
import jax.numpy as jnp
from jax import lax
from jax.experimental import pallas as pl
from jax.experimental.pallas import tpu as pltpu

N_DEV = 8
N_ROUNDS = 3
B, SQ, D = 1, 512, 1024
HQ, HKV, DH = 8, 2, 128
GROUP = HQ // HKV
SCALE = 0.08838834764831843


def kernel(x, Wq, Wo, K_ext, V_ext):
    def body(x_ref, wq_ref, wo_ref, k_ref, v_ref, out_ref,
             state_o, state_ml, send_buf, comm_o, comm_ml,
             send_o, recv_o, send_ml, recv_ml):
        my = lax.axis_index("i")

        barrier = pltpu.get_barrier_semaphore()
        for r in range(N_ROUNDS):
            partner = jnp.bitwise_xor(my, 1 << r)
            pl.semaphore_signal(barrier, inc=1, device_id=(partner,),
                                device_id_type=pl.DeviceIdType.MESH)
        pl.semaphore_wait(barrier, N_ROUNDS)

        q = jnp.dot(x_ref[0], wq_ref[:, :],
                    preferred_element_type=jnp.float32)

        for h in range(HQ):
            qh = q[:, h * DH:(h + 1) * DH]
            kh = k_ref[0, :, h // GROUP, :]
            vh = v_ref[0, :, h // GROUP, :]
            s = lax.dot_general(qh, kh, (((1,), (1,)), ((), ())),
                                preferred_element_type=jnp.float32) * SCALE
            mh = jnp.max(s, axis=1, keepdims=True)
            p = jnp.exp(s - mh)
            lh = jnp.sum(p, axis=1, keepdims=True)
            oh = lax.dot_general(p, vh, (((1,), (0,)), ((), ())),
                                 preferred_element_type=jnp.float32)
            state_o[h, :, :] = oh
            state_ml[0, h, :] = mh[:, 0]
            state_ml[1, h, :] = lh[:, 0]

        for r in range(N_ROUNDS):
            partner = jnp.bitwise_xor(my, 1 << r)
            send_buf[:, :, :] = state_o[:, :, :].astype(jnp.bfloat16)
            rdma_o = pltpu.make_async_remote_copy(
                src_ref=send_buf, dst_ref=comm_o.at[r],
                send_sem=send_o.at[r], recv_sem=recv_o.at[r],
                device_id=(partner,), device_id_type=pl.DeviceIdType.MESH)
            rdma_ml = pltpu.make_async_remote_copy(
                src_ref=state_ml, dst_ref=comm_ml.at[r],
                send_sem=send_ml.at[r], recv_sem=recv_ml.at[r],
                device_id=(partner,), device_id_type=pl.DeviceIdType.MESH)
            rdma_o.start()
            rdma_ml.start()
            rdma_o.wait()
            rdma_ml.wait()

            m_a = state_ml[0, :, :]
            l_a = state_ml[1, :, :]
            o_a = state_o[:, :, :]
            m_b = comm_ml[r, 0, :, :]
            l_b = comm_ml[r, 1, :, :]
            o_b = comm_o[r, :, :, :].astype(jnp.float32)
            m_n = jnp.maximum(m_a, m_b)
            a = jnp.exp(m_a - m_n)
            b = jnp.exp(m_b - m_n)
            state_ml[0, :, :] = m_n
            state_ml[1, :, :] = a * l_a + b * l_b
            state_o[:, :, :] = (a[:, :, None] * o_a + b[:, :, None] * o_b)

        acc = jnp.zeros((SQ, D), jnp.float32)
        for h in range(HQ):
            o_h = state_o[h, :, :] / state_ml[1, h, :][:, None]
            acc = acc + jnp.dot(o_h, wo_ref[h * DH:(h + 1) * DH, :],
                                preferred_element_type=jnp.float32)
        out_ref[0, :, :] = acc

    return pl.pallas_call(
        body,
        out_shape=jax.ShapeDtypeStruct((B, SQ, D), jnp.float32),
        in_specs=[pl.BlockSpec(memory_space=pltpu.VMEM)] * 5,
        out_specs=pl.BlockSpec(memory_space=pltpu.VMEM),
        scratch_shapes=[
            pltpu.VMEM((HQ, SQ, DH), jnp.float32),
            pltpu.VMEM((2, HQ, SQ), jnp.float32),
            pltpu.VMEM((HQ, SQ, DH), jnp.bfloat16),
            pltpu.VMEM((N_ROUNDS, HQ, SQ, DH), jnp.bfloat16),
            pltpu.VMEM((N_ROUNDS, 2, HQ, SQ), jnp.float32),
            pltpu.SemaphoreType.DMA((N_ROUNDS,)),
            pltpu.SemaphoreType.DMA((N_ROUNDS,)),
            pltpu.SemaphoreType.DMA((N_ROUNDS,)),
            pltpu.SemaphoreType.DMA((N_ROUNDS,)),
        ],
        compiler_params=pltpu.CompilerParams(
            collective_id=0, vmem_limit_bytes=64 * 1024 * 1024),
    )(x, Wq, Wo, K_ext, V_ext)


# device time: 64508 ns/iter; 1.8615x vs baseline; 1.3393x over previous
import jax
import jax.numpy as jnp
from jax import lax
from jax.experimental import pallas as pl
from jax.experimental.pallas import tpu as pltpu

N_DEV = 8
N_ROUNDS = 3
B, SQ, D = 1, 512, 1024
HQ, HKV, DH = 8, 2, 128
GROUP = HQ // HKV
GD = GROUP * DH
SCALE = 0.08838834764831843


def kernel(x, Wq, Wo, K_ext, V_ext):
    def body(x_ref, wq_ref, wo_ref, k_ref, v_ref, out_ref,
             state_o, state_ml, send_buf, comm_o, comm_ml,
             send_o, recv_o, send_ml, recv_ml):
        my = lax.axis_index("i")

        barrier = pltpu.get_barrier_semaphore()
        for r in range(N_ROUNDS):
            partner = jnp.bitwise_xor(my, 1 << r)
            pl.semaphore_signal(barrier, inc=1, device_id=(partner,),
                                device_id_type=pl.DeviceIdType.MESH)
        pl.semaphore_wait(barrier, N_ROUNDS)

        def attn_group(g):
            qg = jnp.dot(x_ref[0], wq_ref[:, g * GD:(g + 1) * GD],
                         preferred_element_type=jnp.float32)
            kh = k_ref[0, :, g, :]
            vh = v_ref[0, :, g, :]
            for h in range(GROUP):
                qh = qg[:, h * DH:(h + 1) * DH]
                s = lax.dot_general(qh, kh, (((1,), (1,)), ((), ())),
                                    preferred_element_type=jnp.float32)
                s = s * SCALE
                mh = jnp.max(s, axis=1, keepdims=True)
                p = jnp.exp(s - mh)
                lh = jnp.sum(p, axis=1, keepdims=True)
                oh = lax.dot_general(p, vh, (((1,), (0,)), ((), ())),
                                     preferred_element_type=jnp.float32)
                state_o[g, h, :, :] = oh
                state_ml[g, 0, h, :] = mh[:, 0]
                state_ml[g, 1, h, :] = lh[:, 0]

        def start_round(r, g):
            partner = jnp.bitwise_xor(my, 1 << r)
            send_buf[g, :, :, :] = state_o[g, :, :, :].astype(jnp.bfloat16)
            rdma_o = pltpu.make_async_remote_copy(
                src_ref=send_buf.at[g], dst_ref=comm_o.at[r, g],
                send_sem=send_o.at[r, g], recv_sem=recv_o.at[r, g],
                device_id=(partner,), device_id_type=pl.DeviceIdType.MESH)
            rdma_ml = pltpu.make_async_remote_copy(
                src_ref=state_ml.at[g], dst_ref=comm_ml.at[r, g],
                send_sem=send_ml.at[r, g], recv_sem=recv_ml.at[r, g],
                device_id=(partner,), device_id_type=pl.DeviceIdType.MESH)
            rdma_o.start()
            rdma_ml.start()
            return rdma_o, rdma_ml

        def finish_round(r, g, rdmas):
            rdma_o, rdma_ml = rdmas
            rdma_o.wait()
            rdma_ml.wait()
            m_a = state_ml[g, 0, :, :]
            l_a = state_ml[g, 1, :, :]
            o_a = state_o[g, :, :, :]
            m_b = comm_ml[r, g, 0, :, :]
            l_b = comm_ml[r, g, 1, :, :]
            o_b = comm_o[r, g, :, :, :].astype(jnp.float32)
            m_n = jnp.maximum(m_a, m_b)
            a = jnp.exp(m_a - m_n)
            b = jnp.exp(m_b - m_n)
            state_ml[g, 0, :, :] = m_n
            state_ml[g, 1, :, :] = a * l_a + b * l_b
            state_o[g, :, :, :] = a[:, :, None] * o_a + b[:, :, None] * o_b

        def group_out(g):
            acc = jnp.zeros((SQ, D), jnp.float32)
            for h in range(GROUP):
                hh = g * GROUP + h
                o_h = state_o[g, h, :, :] / state_ml[g, 1, h, :][:, None]
                acc = acc + jnp.dot(o_h, wo_ref[hh * DH:(hh + 1) * DH, :],
                                    preferred_element_type=jnp.float32)
            return acc

        attn_group(0)
        r0a = start_round(0, 0)
        attn_group(1)
        r0b = start_round(0, 1)
        finish_round(0, 0, r0a)
        r1a = start_round(1, 0)
        finish_round(0, 1, r0b)
        r1b = start_round(1, 1)
        finish_round(1, 0, r1a)
        r2a = start_round(2, 0)
        finish_round(1, 1, r1b)
        r2b = start_round(2, 1)
        finish_round(2, 0, r2a)
        acc = group_out(0)
        finish_round(2, 1, r2b)
        out_ref[0, :, :] = acc + group_out(1)

    return pl.pallas_call(
        body,
        out_shape=jax.ShapeDtypeStruct((B, SQ, D), jnp.float32),
        in_specs=[pl.BlockSpec(memory_space=pltpu.VMEM)] * 5,
        out_specs=pl.BlockSpec(memory_space=pltpu.VMEM),
        scratch_shapes=[
            pltpu.VMEM((HKV, GROUP, SQ, DH), jnp.float32),
            pltpu.VMEM((HKV, 2, GROUP, SQ), jnp.float32),
            pltpu.VMEM((HKV, GROUP, SQ, DH), jnp.bfloat16),
            pltpu.VMEM((N_ROUNDS, HKV, GROUP, SQ, DH), jnp.bfloat16),
            pltpu.VMEM((N_ROUNDS, HKV, 2, GROUP, SQ), jnp.float32),
            pltpu.SemaphoreType.DMA((N_ROUNDS, HKV)),
            pltpu.SemaphoreType.DMA((N_ROUNDS, HKV)),
            pltpu.SemaphoreType.DMA((N_ROUNDS, HKV)),
            pltpu.SemaphoreType.DMA((N_ROUNDS, HKV)),
        ],
        compiler_params=pltpu.CompilerParams(
            collective_id=0, vmem_limit_bytes=64 * 1024 * 1024),
    )(x, Wq, Wo, K_ext, V_ext)
